# Initial kernel scaffold; baseline (speedup 1.0000x reference)
import jax
import jax.numpy as jnp
from jax.experimental import pallas as pl


def kernel(x, edge_index, W_g0, b_g0, W_g1, b_g1, W_l0, b_l0, W_l1, b_l1):
    raise NotImplementedError("write your pallas kernel here")


# speedup vs baseline: 1.4421x; 1.4421x over previous
import functools

import jax
import jax.numpy as jnp
from jax import lax
from jax.experimental import pallas as pl
from jax.experimental.pallas import tpu as pltpu
from jax.experimental.pallas import tpu_sc as plsc

N = 10000
E = 320000
D = 128

NW = 32
L_CHUNK = 128
CH = 79
EPW = CH * L_CHUNK
E_PAD = NW * EPW
N_PAD = 10240

EC = 2048
NB = 1024
N_EC = E_PAD // EC
N_NB = N_PAD // NB

_mesh = plsc.VectorSubcoreMesh(core_axis_name="c", subcore_axis_name="s")


def _gather_body(g_hbm, row_hbm, out_hbm, rowv, rows, sem):
    c = lax.axis_index("c")
    s = lax.axis_index("s")
    wid = s * 2 + c
    base = wid * EPW
    pltpu.sync_copy(row_hbm.at[wid], rowv)

    def chunk(j, _):
        pltpu.async_copy(g_hbm.at[rowv.at[j]], rows, sem).wait()
        pltpu.sync_copy(rows, out_hbm.at[pl.ds(base + j * L_CHUNK, L_CHUNK)])
        return 0

    lax.fori_loop(0, CH, chunk, 0)


_gather_call = functools.partial(
    pl.kernel,
    _gather_body,
    out_type=jax.ShapeDtypeStruct((E_PAD, D), jnp.float32),
    mesh=_mesh,
    scratch_types=[
        pltpu.VMEM((CH, L_CHUNK), jnp.int32),
        pltpu.VMEM((L_CHUNK, D), jnp.float32),
        pltpu.SemaphoreType.DMA,
    ],
)()


def _scat_body(col_ref, g_ref, out_ref):
    n = pl.program_id(0)
    e = pl.program_id(1)

    @pl.when(e == 0)
    def _():
        out_ref[...] = jnp.zeros_like(out_ref)

    cb = col_ref[0, 0, :]
    node_ids = lax.broadcasted_iota(jnp.int32, (NB, EC), 0) + n * NB
    onehot = (node_ids == cb[None, :]).astype(jnp.bfloat16)
    out_ref[...] += jnp.dot(
        onehot, g_ref[...].astype(jnp.bfloat16), preferred_element_type=jnp.float32
    )


_scat_call = pl.pallas_call(
    _scat_body,
    grid=(N_NB, N_EC),
    in_specs=[
        pl.BlockSpec((1, 1, EC), lambda n, e: (e, 0, 0)),
        pl.BlockSpec((EC, D), lambda n, e: (e, 0)),
    ],
    out_specs=pl.BlockSpec((NB, D), lambda n, e: (n, 0)),
    out_shape=jax.ShapeDtypeStruct((N_PAD, D), jnp.float32),
)


def _deg_body(col_ref, out_ref):
    n = pl.program_id(0)
    e = pl.program_id(1)

    @pl.when(e == 0)
    def _():
        out_ref[...] = jnp.zeros_like(out_ref)

    cb = col_ref[0, 0, :]
    node_ids = lax.broadcasted_iota(jnp.int32, (NB, EC), 0) + n * NB
    onehot = (node_ids == cb[None, :]).astype(jnp.bfloat16)
    ones = jnp.ones((EC, 8), jnp.bfloat16)
    out_ref[...] += jnp.dot(onehot, ones, preferred_element_type=jnp.float32)


_deg_call = pl.pallas_call(
    _deg_body,
    grid=(N_NB, N_EC),
    in_specs=[pl.BlockSpec((1, 1, EC), lambda n, e: (e, 0, 0))],
    out_specs=pl.BlockSpec((NB, 8), lambda n, e: (n, 0)),
    out_shape=jax.ShapeDtypeStruct((N_PAD, 8), jnp.float32),
)


def _dinv(degp):
    return lax.rsqrt(degp[:N, 0:1] + 1.0)


def _tc_g1_body(x_ref, w_ref, degp_ref, g1_ref):
    dinv = _dinv(degp_ref[...])
    g1_ref[...] = dinv * jnp.dot(
        x_ref[...], w_ref[...], preferred_element_type=jnp.float32
    )


def _tc_mid_body(s1_ref, g1_ref, degp_ref, b_ref, w_ref, g2_ref):
    dinv = _dinv(degp_ref[...])
    h1 = dinv * (s1_ref[:N, :] + g1_ref[...]) + b_ref[...]
    g2_ref[...] = dinv * jnp.dot(h1, w_ref[...], preferred_element_type=jnp.float32)


def _tc_out_body(s2_ref, g2_ref, degp_ref, b_ref, w0_ref, b0_ref, w1_ref, b1_ref, out_ref):
    dinv = _dinv(degp_ref[...])
    h2 = dinv * (s2_ref[:N, :] + g2_ref[...]) + b_ref[...]
    t = jnp.dot(h2, w0_ref[...].T, preferred_element_type=jnp.float32) + b0_ref[...]
    out_ref[...] = (
        jnp.dot(t, w1_ref[...].T, preferred_element_type=jnp.float32) + b1_ref[...]
    )


_tc_g1 = pl.pallas_call(
    _tc_g1_body, out_shape=jax.ShapeDtypeStruct((N, D), jnp.float32)
)
_tc_mid = pl.pallas_call(
    _tc_mid_body, out_shape=jax.ShapeDtypeStruct((N, D), jnp.float32)
)
_tc_out = pl.pallas_call(
    _tc_out_body, out_shape=jax.ShapeDtypeStruct((N, 5), jnp.float32)
)


def kernel(x, edge_index, W_g0, b_g0, W_g1, b_g1, W_l0, b_l0, W_l1, b_l1):
    row = edge_index[0]
    col = edge_index[1]
    pad = E_PAD - E
    row3 = jnp.concatenate([row, jnp.zeros((pad,), row.dtype)]).reshape(NW, CH, L_CHUNK)
    colp = jnp.concatenate([col, jnp.full((pad,), N_PAD - 1, col.dtype)])
    col3 = colp.reshape(N_EC, 1, EC)

    degp = _deg_call(col3)
    g1 = _tc_g1(x, W_g0, degp)
    G1 = _gather_call(g1, row3)
    s1 = _scat_call(col3, G1)
    g2 = _tc_mid(s1, g1, degp, b_g0, W_g1)
    G2 = _gather_call(g2, row3)
    s2 = _scat_call(col3, G2)
    return _tc_out(s2, g2, degp, b_g1, W_l0, b_l0, W_l1, b_l1)
